# baseline (speedup 1.0000x reference)
import functools

import jax
import jax.numpy as jnp
from jax import lax
from jax.experimental import pallas as pl
from jax.experimental.pallas import tpu as pltpu
from jax.experimental.pallas import tpu_sc as plsc

N = 10000
E = 320000
D_IN = 128
H = 32
D_OUT = 128

NC, NS = 2, 16
NW = NC * NS
CHUNK = 128
NBUF = 4
N_PAD = 10240
ROWS_PER_TILE = N_PAD // NS
EPT = -(-E // (NW * CHUNK * NBUF)) * CHUNK * NBUF
NCHUNK = EPT // CHUNK
E_PAD = EPT * NW

_sc_mesh = plsc.VectorSubcoreMesh(
    core_axis_name="c", subcore_axis_name="s", num_cores=NC, num_subcores=NS)


def _agg_body(feat_hbm, src_hbm, dst_hbm, out_hbm,
              src_v, dst_v, rb0, rb1, rb2, rb3, iobuf, acc_sh,
              sm0, sm1, sm2, sm3):
    bufs = (rb0, rb1, rb2, rb3)
    sems = (sm0, sm1, sm2, sm3)
    c = lax.axis_index("c")
    s = lax.axis_index("s")
    wid = s * NC + c

    z = jnp.zeros((16,), jnp.float32)

    def zrow(i, _):
        iobuf[i, pl.ds(0, 16)] = z
        iobuf[i, pl.ds(16, 16)] = z
        return 0

    lax.fori_loop(0, ROWS_PER_TILE, zrow, 0)
    rows_sl = pl.ds(s * ROWS_PER_TILE, ROWS_PER_TILE)
    pltpu.sync_copy(iobuf, acc_sh.at[rows_sl])

    pltpu.sync_copy(src_hbm.at[wid], src_v)
    pltpu.sync_copy(dst_hbm.at[wid], dst_v)
    plsc.subcore_barrier()

    for b in range(NBUF):
        pltpu.async_copy(feat_hbm.at[src_v.at[b]], bufs[b], sems[b])

    def body(i, _):
        j0 = i * NBUF
        for b in range(NBUF):
            pltpu.make_async_copy(feat_hbm.at[src_v.at[b]], bufs[b],
                                  sems[b]).wait()
            pltpu.sync_copy(bufs[b], acc_sh.at[dst_v.at[j0 + b]], add=True)
            pltpu.async_copy(feat_hbm.at[src_v.at[j0 + b + NBUF]], bufs[b],
                             sems[b])
        return 0

    lax.fori_loop(0, NCHUNK // NBUF - 1, body, 0)
    for b in range(NBUF):
        j = NCHUNK - NBUF + b
        pltpu.make_async_copy(feat_hbm.at[src_v.at[b]], bufs[b],
                              sems[b]).wait()
        pltpu.sync_copy(bufs[b], acc_sh.at[dst_v.at[j]], add=True)
    plsc.subcore_barrier()

    pltpu.sync_copy(acc_sh.at[rows_sl], iobuf)
    pltpu.sync_copy(iobuf, out_hbm.at[c].at[rows_sl])


_agg = pl.kernel(
    _agg_body,
    out_type=jax.ShapeDtypeStruct((NC, N_PAD, H), jnp.float32),
    mesh=_sc_mesh,
    scratch_types=[
        pltpu.VMEM((NCHUNK, CHUNK), jnp.int32),
        pltpu.VMEM((NCHUNK, CHUNK), jnp.int32),
        pltpu.VMEM((CHUNK, H), jnp.float32),
        pltpu.VMEM((CHUNK, H), jnp.float32),
        pltpu.VMEM((CHUNK, H), jnp.float32),
        pltpu.VMEM((CHUNK, H), jnp.float32),
        pltpu.VMEM((ROWS_PER_TILE, H), jnp.float32),
        pltpu.VMEM_SHARED((N_PAD, H), jnp.float32),
        pltpu.SemaphoreType.DMA,
        pltpu.SemaphoreType.DMA,
        pltpu.SemaphoreType.DMA,
        pltpu.SemaphoreType.DMA,
    ],
    compiler_params=pltpu.CompilerParams(use_tc_tiling_on_sc=False),
)


def _mm1_body(x_ref, w_ref, o_ref):
    o_ref[...] = jnp.dot(x_ref[...], w_ref[...],
                         preferred_element_type=jnp.float32)


def _mlp_body(y_ref, a0_ref, a1_ref, w2_ref, b1_ref, b2_ref, o_ref):
    h = jnp.maximum(y_ref[...] + a0_ref[...] + a1_ref[...] + b1_ref[...], 0.0)
    h = jnp.dot(h, w2_ref[...], preferred_element_type=jnp.float32) + b2_ref[...]
    o_ref[...] = jnp.maximum(h, 0.0)


def _out_body(h_ref, g0_ref, g1_ref, w3_ref, b3_ref, o_ref):
    h = h_ref[...] + g0_ref[...] + g1_ref[...]
    h = jnp.dot(h, w3_ref[...], preferred_element_type=jnp.float32) + b3_ref[...]
    o_ref[...] = jnp.maximum(h, 0.0)


_mm1 = pl.pallas_call(
    _mm1_body, out_shape=jax.ShapeDtypeStruct((N, H), jnp.float32))
_mlp = pl.pallas_call(
    _mlp_body, out_shape=jax.ShapeDtypeStruct((N, H), jnp.float32))
_out = pl.pallas_call(
    _out_body, out_shape=jax.ShapeDtypeStruct((N, D_OUT), jnp.float32))


def kernel(x, edge_index, W1, b1, W2, b2, W3, b3):
    src = edge_index[0]
    dst = edge_index[1]
    pad = E_PAD - E
    src_p = jnp.concatenate(
        [src, jnp.zeros((pad,), jnp.int32)]).reshape(NW, NCHUNK, CHUNK)
    dst_p = jnp.concatenate(
        [dst, jnp.full((pad,), N, jnp.int32)]).reshape(NW, NCHUNK, CHUNK)

    y = _mm1(x, W1)
    agg1 = _agg(y, src_p, dst_p)
    h4 = _mlp(y, agg1[0, :N], agg1[1, :N], W2,
              b1.reshape(1, H), b2.reshape(1, H))
    agg2 = _agg(h4, src_p, dst_p)
    out = _out(h4, agg2[0, :N], agg2[1, :N], W3, b3.reshape(1, D_OUT))
    return out


# speedup vs baseline: 2.1059x; 2.1059x over previous
import functools

import jax
import jax.numpy as jnp
from jax import lax
from jax.experimental import pallas as pl
from jax.experimental.pallas import tpu as pltpu
from jax.experimental.pallas import tpu_sc as plsc

N = 10000
E = 320000
D_IN = 128
H = 32
D_OUT = 128

NC, NS = 2, 16
NW = NC * NS
CHUNK = 128
NBUF = 4
N_PAD = 10240
ROWS_PER_TILE = N_PAD // NS
EPT = -(-E // (NW * CHUNK * NBUF)) * CHUNK * NBUF
NCHUNK = EPT // CHUNK
E_PAD = EPT * NW

_sc_mesh = plsc.VectorSubcoreMesh(
    core_axis_name="c", subcore_axis_name="s", num_cores=NC, num_subcores=NS)


def _agg_body(feat_hbm, src_hbm, dst_hbm, out_hbm,
              src_v, dst_v, rb0, rb1, rb2, rb3, iobuf, acc_sh, feat_sh,
              sm0, sm1, sm2, sm3):
    bufs = (rb0, rb1, rb2, rb3)
    sems = (sm0, sm1, sm2, sm3)
    c = lax.axis_index("c")
    s = lax.axis_index("s")
    wid = s * NC + c

    feat_sl = pl.ds(s * (N // NS), N // NS)
    pltpu.sync_copy(feat_hbm.at[feat_sl], feat_sh.at[feat_sl])

    z = jnp.zeros((16,), jnp.float32)

    def zrow(i, _):
        iobuf[i, pl.ds(0, 16)] = z
        iobuf[i, pl.ds(16, 16)] = z
        return 0

    lax.fori_loop(0, ROWS_PER_TILE, zrow, 0)
    rows_sl = pl.ds(s * ROWS_PER_TILE, ROWS_PER_TILE)
    pltpu.sync_copy(iobuf, acc_sh.at[rows_sl])

    pltpu.sync_copy(src_hbm.at[wid], src_v)
    pltpu.sync_copy(dst_hbm.at[wid], dst_v)
    plsc.subcore_barrier()

    for b in range(NBUF):
        pltpu.async_copy(feat_sh.at[src_v.at[b]], bufs[b], sems[b])

    def body(i, _):
        j0 = i * NBUF
        for b in range(NBUF):
            pltpu.make_async_copy(feat_sh.at[src_v.at[b]], bufs[b],
                                  sems[b]).wait()
            pltpu.sync_copy(bufs[b], acc_sh.at[dst_v.at[j0 + b]], add=True)
            pltpu.async_copy(feat_sh.at[src_v.at[j0 + b + NBUF]], bufs[b],
                             sems[b])
        return 0

    lax.fori_loop(0, NCHUNK // NBUF - 1, body, 0)
    for b in range(NBUF):
        j = NCHUNK - NBUF + b
        pltpu.make_async_copy(feat_sh.at[src_v.at[b]], bufs[b],
                              sems[b]).wait()
        pltpu.sync_copy(bufs[b], acc_sh.at[dst_v.at[j]], add=True)
    plsc.subcore_barrier()

    pltpu.sync_copy(acc_sh.at[rows_sl], iobuf)
    pltpu.sync_copy(iobuf, out_hbm.at[c].at[rows_sl])


_agg = pl.kernel(
    _agg_body,
    out_type=jax.ShapeDtypeStruct((NC, N_PAD, H), jnp.float32),
    mesh=_sc_mesh,
    scratch_types=[
        pltpu.VMEM((NCHUNK, CHUNK), jnp.int32),
        pltpu.VMEM((NCHUNK, CHUNK), jnp.int32),
        pltpu.VMEM((CHUNK, H), jnp.float32),
        pltpu.VMEM((CHUNK, H), jnp.float32),
        pltpu.VMEM((CHUNK, H), jnp.float32),
        pltpu.VMEM((CHUNK, H), jnp.float32),
        pltpu.VMEM((ROWS_PER_TILE, H), jnp.float32),
        pltpu.VMEM_SHARED((N_PAD, H), jnp.float32),
        pltpu.VMEM_SHARED((N, H), jnp.float32),
        pltpu.SemaphoreType.DMA,
        pltpu.SemaphoreType.DMA,
        pltpu.SemaphoreType.DMA,
        pltpu.SemaphoreType.DMA,
    ],
    compiler_params=pltpu.CompilerParams(use_tc_tiling_on_sc=False),
)


def _mm1_body(x_ref, w_ref, o_ref):
    o_ref[...] = jnp.dot(x_ref[...], w_ref[...],
                         preferred_element_type=jnp.float32)


def _mlp_body(y_ref, a0_ref, a1_ref, w2_ref, b1_ref, b2_ref, o_ref):
    h = jnp.maximum(y_ref[...] + a0_ref[...] + a1_ref[...] + b1_ref[...], 0.0)
    h = jnp.dot(h, w2_ref[...], preferred_element_type=jnp.float32) + b2_ref[...]
    o_ref[...] = jnp.maximum(h, 0.0)


def _out_body(h_ref, g0_ref, g1_ref, w3_ref, b3_ref, o_ref):
    h = h_ref[...] + g0_ref[...] + g1_ref[...]
    h = jnp.dot(h, w3_ref[...], preferred_element_type=jnp.float32) + b3_ref[...]
    o_ref[...] = jnp.maximum(h, 0.0)


_mm1 = pl.pallas_call(
    _mm1_body, out_shape=jax.ShapeDtypeStruct((N, H), jnp.float32))
_mlp = pl.pallas_call(
    _mlp_body, out_shape=jax.ShapeDtypeStruct((N, H), jnp.float32))
_out = pl.pallas_call(
    _out_body, out_shape=jax.ShapeDtypeStruct((N, D_OUT), jnp.float32))


def kernel(x, edge_index, W1, b1, W2, b2, W3, b3):
    src = edge_index[0]
    dst = edge_index[1]
    pad = E_PAD - E
    src_p = jnp.concatenate(
        [src, jnp.zeros((pad,), jnp.int32)]).reshape(NW, NCHUNK, CHUNK)
    dst_p = jnp.concatenate(
        [dst, jnp.full((pad,), N, jnp.int32)]).reshape(NW, NCHUNK, CHUNK)

    y = _mm1(x, W1)
    agg1 = _agg(y, src_p, dst_p)
    h4 = _mlp(y, agg1[0, :N], agg1[1, :N], W2,
              b1.reshape(1, H), b2.reshape(1, H))
    agg2 = _agg(h4, src_p, dst_p)
    out = _out(h4, agg2[0, :N], agg2[1, :N], W3, b3.reshape(1, D_OUT))
    return out


# speedup vs baseline: 2.2835x; 1.0843x over previous
import functools

import jax
import jax.numpy as jnp
from jax import lax
from jax.experimental import pallas as pl
from jax.experimental.pallas import tpu as pltpu
from jax.experimental.pallas import tpu_sc as plsc

N = 10000
E = 320000
D_IN = 128
H = 32
D_OUT = 128

NC, NS = 2, 16
NW = NC * NS
CHUNK = 128
NBUF = 4
N_PAD = 10240
ROWS_PER_TILE = N_PAD // NS
EPT = -(-E // (NW * CHUNK * NBUF)) * CHUNK * NBUF
NCHUNK = EPT // CHUNK
E_PAD = EPT * NW

_sc_mesh = plsc.VectorSubcoreMesh(
    core_axis_name="c", subcore_axis_name="s", num_cores=NC, num_subcores=NS)


def _agg_body(feat_hbm, src_hbm, dst_hbm, out_hbm,
              src_v, dst_v, rb0, rb1, rb2, rb3, iobuf, acc_sh, feat_sh,
              sm0, sm1, sm2, sm3):
    bufs = (rb0, rb1, rb2, rb3)
    sems = (sm0, sm1, sm2, sm3)
    c = lax.axis_index("c")
    s = lax.axis_index("s")
    wid = s * NC + c

    feat_sl = pl.ds(s * (N // NS), N // NS)
    pltpu.sync_copy(feat_hbm.at[feat_sl], feat_sh.at[feat_sl])

    z = jnp.zeros((16,), jnp.float32)

    def zrow(i, _):
        iobuf[i, pl.ds(0, 16)] = z
        iobuf[i, pl.ds(16, 16)] = z
        return 0

    lax.fori_loop(0, ROWS_PER_TILE, zrow, 0)
    rows_sl = pl.ds(s * ROWS_PER_TILE, ROWS_PER_TILE)
    pltpu.sync_copy(iobuf, acc_sh.at[rows_sl])

    pltpu.sync_copy(src_hbm.at[wid], src_v)
    pltpu.sync_copy(dst_hbm.at[wid], dst_v)
    plsc.subcore_barrier()

    for b in range(NBUF):
        pltpu.async_copy(feat_sh.at[src_v.at[b]], bufs[b], sems[b])

    def body(i, _):
        j0 = i * NBUF
        for b in range(NBUF):
            pltpu.make_async_copy(feat_sh.at[src_v.at[b]], bufs[b],
                                  sems[b]).wait()
            pltpu.sync_copy(bufs[b], acc_sh.at[dst_v.at[j0 + b]], add=True)
            pltpu.async_copy(feat_sh.at[src_v.at[j0 + b + NBUF]], bufs[b],
                             sems[b])
        return 0

    lax.fori_loop(0, NCHUNK // NBUF - 1, body, 0)
    for b in range(NBUF):
        j = NCHUNK - NBUF + b
        pltpu.make_async_copy(feat_sh.at[src_v.at[b]], bufs[b],
                              sems[b]).wait()
        pltpu.sync_copy(bufs[b], acc_sh.at[dst_v.at[j]], add=True)
    plsc.subcore_barrier()

    pltpu.sync_copy(acc_sh.at[rows_sl], iobuf)
    pltpu.sync_copy(iobuf, out_hbm.at[c].at[rows_sl])


_agg = pl.kernel(
    _agg_body,
    out_type=jax.ShapeDtypeStruct((NC, N_PAD, H), jnp.float32),
    mesh=_sc_mesh,
    scratch_types=[
        pltpu.VMEM((NCHUNK, CHUNK), jnp.int32),
        pltpu.VMEM((NCHUNK, CHUNK), jnp.int32),
        pltpu.VMEM((CHUNK, H), jnp.float32),
        pltpu.VMEM((CHUNK, H), jnp.float32),
        pltpu.VMEM((CHUNK, H), jnp.float32),
        pltpu.VMEM((CHUNK, H), jnp.float32),
        pltpu.VMEM((ROWS_PER_TILE, H), jnp.float32),
        pltpu.VMEM_SHARED((N_PAD, H), jnp.float32),
        pltpu.VMEM_SHARED((N, H), jnp.float32),
        pltpu.SemaphoreType.DMA,
        pltpu.SemaphoreType.DMA,
        pltpu.SemaphoreType.DMA,
        pltpu.SemaphoreType.DMA,
    ],
    compiler_params=pltpu.CompilerParams(use_tc_tiling_on_sc=False),
)


def _mm1_body(x_ref, w_ref, o_ref):
    o_ref[...] = jnp.dot(x_ref[...], w_ref[...],
                         preferred_element_type=jnp.float32)


def _mlp_body(y_ref, agg_ref, w2_ref, b1_ref, b2_ref, o_ref):
    a = agg_ref[0, 0:N, :] + agg_ref[1, 0:N, :]
    h = jnp.maximum(y_ref[...] + a + b1_ref[...], 0.0)
    h = jnp.dot(h, w2_ref[...], preferred_element_type=jnp.float32) + b2_ref[...]
    o_ref[...] = jnp.maximum(h, 0.0)


def _out_body(h_ref, agg_ref, w3_ref, b3_ref, o_ref):
    h = h_ref[...] + agg_ref[0, 0:N, :] + agg_ref[1, 0:N, :]
    h = jnp.dot(h, w3_ref[...], preferred_element_type=jnp.float32) + b3_ref[...]
    o_ref[...] = jnp.maximum(h, 0.0)


_mm1 = pl.pallas_call(
    _mm1_body, out_shape=jax.ShapeDtypeStruct((N, H), jnp.float32))
_mlp = pl.pallas_call(
    _mlp_body, out_shape=jax.ShapeDtypeStruct((N, H), jnp.float32))
_out = pl.pallas_call(
    _out_body, out_shape=jax.ShapeDtypeStruct((N, D_OUT), jnp.float32))


def kernel(x, edge_index, W1, b1, W2, b2, W3, b3):
    src = edge_index[0]
    dst = edge_index[1]
    pad = E_PAD - E
    src_p = jnp.concatenate(
        [src, jnp.zeros((pad,), jnp.int32)]).reshape(NW, NCHUNK, CHUNK)
    dst_p = jnp.concatenate(
        [dst, jnp.full((pad,), N, jnp.int32)]).reshape(NW, NCHUNK, CHUNK)

    y = _mm1(x, W1)
    agg1 = _agg(y, src_p, dst_p)
    h4 = _mlp(y, agg1, W2, b1.reshape(1, H), b2.reshape(1, H))
    agg2 = _agg(h4, src_p, dst_p)
    out = _out(h4, agg2, W3, b3.reshape(1, D_OUT))
    return out
